# baseline (speedup 1.0000x reference)
import functools

import jax
import jax.numpy as jnp
from jax import lax
from jax.experimental import pallas as pl
from jax.experimental.pallas import tpu as pltpu
from jax.experimental.pallas import tpu_sc as plsc

EMB = 32
CHUNK = 128


@functools.lru_cache(maxsize=None)
def _build_gather(batch):
    nc, ns = 2, 16
    nw = nc * ns
    per_w = batch // nw
    n_chunks = per_w // CHUNK
    mesh = plsc.VectorSubcoreMesh(core_axis_name="c", subcore_axis_name="s")

    @functools.partial(
        pl.kernel,
        mesh=mesh,
        compiler_params=pltpu.CompilerParams(
            needs_layout_passes=False, use_tc_tiling_on_sc=False),
        out_type=(
            jax.ShapeDtypeStruct((batch, EMB), jnp.float32),
            jax.ShapeDtypeStruct((batch, EMB), jnp.float32),
            jax.ShapeDtypeStruct((batch,), jnp.float32),
            jax.ShapeDtypeStruct((batch,), jnp.float32),
        ),
        scratch_types=[
            pltpu.VMEM((512,), jnp.int32),
            pltpu.VMEM((512,), jnp.int32),
            pltpu.VMEM((512, EMB), jnp.float32),
            pltpu.VMEM((512, EMB), jnp.float32),
            pltpu.VMEM((512,), jnp.float32),
            pltpu.VMEM((512,), jnp.float32),
            pltpu.SemaphoreType.DMA,
        ],
    )
    def k(uemb, memb, ubias, mbias, uids, mids,
          ue_hbm, me_hbm, ub_hbm, mb_hbm,
          uid_v, mid_v, ue_v, me_v, ub_v, mb_v, sem):
        wid = lax.axis_index("s") * nc + lax.axis_index("c")
        base = wid * per_w
        pltpu.sync_copy(uids.at[pl.ds(base, per_w)], uid_v)
        pltpu.sync_copy(mids.at[pl.ds(base, per_w)], mid_v)

        copies = []
        for c in range(n_chunks):
            s = pl.ds(c * CHUNK, CHUNK)
            copies.append(pltpu.async_copy(
                uemb.at[uid_v.at[s]], ue_v.at[s], sem))
            copies.append(pltpu.async_copy(
                memb.at[mid_v.at[s]], me_v.at[s], sem))
            copies.append(pltpu.async_copy(
                ubias.at[uid_v.at[s]], ub_v.at[s], sem))
            copies.append(pltpu.async_copy(
                mbias.at[mid_v.at[s]], mb_v.at[s], sem))
        for cp in copies:
            cp.wait()

        pltpu.sync_copy(ue_v, ue_hbm.at[pl.ds(base, per_w)])
        pltpu.sync_copy(me_v, me_hbm.at[pl.ds(base, per_w)])
        pltpu.sync_copy(ub_v, ub_hbm.at[pl.ds(base, per_w)])
        pltpu.sync_copy(mb_v, mb_hbm.at[pl.ds(base, per_w)])

    return k


def _dot_body(ue_ref, me_ref, ub_ref, mb_ref, out_ref):
    dot = jnp.sum(ue_ref[...] * me_ref[...], axis=1)
    out_ref[...] = dot + ub_ref[...] + mb_ref[...]


@functools.lru_cache(maxsize=None)
def _build_dot(batch):
    return pl.pallas_call(
        _dot_body,
        out_shape=jax.ShapeDtypeStruct((batch,), jnp.float32),
    )


def kernel(user_ids, movie_ids, user_emb, movie_emb, user_bias, movie_bias):
    batch = user_ids.shape[0]
    gather = _build_gather(batch)
    ue, me, ub, mb = gather(
        user_emb, movie_emb,
        user_bias.reshape(-1), movie_bias.reshape(-1),
        user_ids.astype(jnp.int32), movie_ids.astype(jnp.int32))
    return _build_dot(batch)(ue, me, ub, mb)
